# baseline (speedup 1.0000x reference)
import functools

import jax
import jax.numpy as jnp
import numpy as np
from jax.experimental import pallas as pl
from jax.experimental.pallas import tpu as pltpu
from jax.sharding import PartitionSpec as P

DIM_STATE = 32
N_PARTICLES = 1024
DIM_OBS = 32
HIDDEN = 64
BATCH = 64
SEQ = 16

NDEV = 2
B_LOC = BATCH // NDEV
ROWS_L = B_LOC * N_PARTICLES
BLK = 2048

_TINY = np.float32(np.finfo(np.float32).tiny)
_LO = np.float32(np.nextafter(np.float32(-1.0), np.float32(0.0)))


def _threefry_xor(kd0, kd1, x1):
    ks0 = kd0
    ks1 = kd1
    ks2 = ks0 ^ ks1 ^ jnp.uint32(0x1BD11BDA)
    x0 = jnp.zeros_like(x1) + ks0
    x1 = x1 + ks1
    rots = ((13, 15, 26, 6), (17, 29, 16, 24))
    ks = (ks0, ks1, ks2)

    def rotl(x, d):
        return (x << jnp.uint32(d)) | (x >> jnp.uint32(32 - d))

    for i in range(5):
        for r in rots[i % 2]:
            x0 = x0 + x1
            x1 = rotl(x1, r)
            x1 = x0 ^ x1
        x0 = x0 + ks[(i + 1) % 3]
        x1 = x1 + ks[(i + 2) % 3] + jnp.uint32(i + 1)
    return x0 ^ x1


def _bits_to_unit(bits):
    fb = (bits >> jnp.uint32(9)) | jnp.uint32(0x3F800000)
    return jax.lax.bitcast_convert_type(fb, jnp.float32) - jnp.float32(1.0)


PBLK = 128
PCHUNK = 4


def _cat_kernel(k1_ref, b0_ref, w_ref, idx_ref):
    i = pl.program_id(0)
    wv = w_ref[...]
    ks0 = k1_ref[0, 0]
    ks1 = k1_ref[0, 1]
    b0 = b0_ref[0, 0]
    Pn = N_PARTICLES

    bb = jax.lax.broadcasted_iota(jnp.uint32, (PCHUNK, B_LOC, Pn), 1)
    pp = jax.lax.broadcasted_iota(jnp.uint32, (PCHUNK, B_LOC, Pn), 0)
    jj = jax.lax.broadcasted_iota(jnp.uint32, (PCHUNK, B_LOC, Pn), 2)
    jn = jax.lax.broadcasted_iota(jnp.int32, (PCHUNK, B_LOC, Pn), 2)
    b0u = b0.astype(jnp.uint32)

    def body(c, _):
        base_p = (i * PBLK + c * PCHUNK).astype(jnp.uint32)
        cnt = (((base_p + pp) * jnp.uint32(BATCH) + bb + b0u)
               << jnp.uint32(10)) + jj
        bits = _threefry_xor(ks0, ks1, cnt)
        f = _bits_to_unit(bits)
        u = jnp.maximum(_TINY, f * (jnp.float32(1.0) - _TINY) + _TINY)
        val = -jnp.log(-jnp.log(u)) + wv[None, :, :]
        m = jnp.max(val, axis=2, keepdims=True)
        cand = jnp.where(val == m, jn, jnp.int32(Pn))
        idxp = jnp.min(cand, axis=2).astype(jnp.int32)
        idx_ref[pl.ds(c * PCHUNK, PCHUNK), :] = idxp
        return 0

    jax.lax.fori_loop(0, PBLK // PCHUNK, body, 0)


def _cat_pallas(w, k1, b0):
    return pl.pallas_call(
        _cat_kernel,
        grid=(N_PARTICLES // PBLK,),
        in_specs=[
            pl.BlockSpec(memory_space=pltpu.SMEM),
            pl.BlockSpec(memory_space=pltpu.SMEM),
            pl.BlockSpec((B_LOC, N_PARTICLES), lambda i: (0, 0)),
        ],
        out_specs=pl.BlockSpec((PBLK, B_LOC), lambda i: (i, 0)),
        out_shape=jax.ShapeDtypeStruct((N_PARTICLES, B_LOC), jnp.int32),
    )(k1, b0, w)


def _step_kernel(x_ref, st_ref,
                 W1_ref, U1_ref, b1_ref, W2_ref, U2_ref, b2_ref,
                 Wm1_ref, bm1_ref, Wm2_ref, bm2_ref,
                 sto_ref, wo_ref):
    x = x_ref[...]
    st = st_ref[...]
    D = DIM_STATE

    def bdot(a, bmat):
        return jnp.dot(a.astype(jnp.bfloat16), bmat.astype(jnp.bfloat16),
                       preferred_element_type=jnp.float32)

    def lstm(xv, h, c, W, U, b):
        z = bdot(xv, W)
        z = z + bdot(h, U)
        z = z + b[None, :]
        i = jax.nn.sigmoid(z[:, 0:D])
        f = jax.nn.sigmoid(z[:, D:2 * D])
        g = jnp.tanh(z[:, 2 * D:3 * D])
        o = jax.nn.sigmoid(z[:, 3 * D:4 * D])
        c_new = f * c + i * g
        h_new = o * jnp.tanh(c_new)
        return h_new, c_new

    h1f, c1f = lstm(x, st[:, 0:D], st[:, D:2 * D], W1_ref[...], U1_ref[...], b1_ref[...][0])
    h2f, c2f = lstm(h1f, st[:, 2 * D:3 * D], st[:, 3 * D:4 * D],
                    W2_ref[...], U2_ref[...], b2_ref[...][0])
    sto_ref[...] = jnp.concatenate([h1f, c1f, h2f, c2f], axis=1)
    ob = x[:, D:D + DIM_OBS]
    minp = jnp.concatenate([ob, h2f], axis=1)
    hid = jnp.dot(minp.astype(jnp.bfloat16), Wm1_ref[...].astype(jnp.bfloat16),
                  preferred_element_type=jnp.float32)
    hid = jax.nn.relu(hid + bm1_ref[...][0])
    wv = jnp.dot(hid.astype(jnp.bfloat16), Wm2_ref[...].astype(jnp.bfloat16),
                 preferred_element_type=jnp.float32)
    wo_ref[...] = wv + bm2_ref[...][0, 0]


def _row_spec(width):
    return pl.BlockSpec((BLK, width), lambda i: (i, 0))


def _full_spec(shape):
    return pl.BlockSpec(shape, lambda i: tuple(0 for _ in shape))


def _step_pallas(x, st, W1, U1, b1, W2, U2, b2, Wm1, bm1, Wm2, bm2):
    n = ROWS_L // BLK
    out_shapes = [jax.ShapeDtypeStruct((ROWS_L, 4 * DIM_STATE), jnp.float32),
                  jax.ShapeDtypeStruct((ROWS_L, 1), jnp.float32)]
    in_specs = [
        _row_spec(DIM_STATE + DIM_OBS),
        _row_spec(4 * DIM_STATE),
        _full_spec(W1.shape), _full_spec(U1.shape), _full_spec((1, 4 * DIM_STATE)),
        _full_spec(W2.shape), _full_spec(U2.shape), _full_spec((1, 4 * DIM_STATE)),
        _full_spec(Wm1.shape), _full_spec((1, HIDDEN)),
        _full_spec(Wm2.shape), _full_spec((1, 1)),
    ]
    out_specs = [_row_spec(4 * DIM_STATE), _row_spec(1)]
    return pl.pallas_call(
        _step_kernel,
        grid=(n,),
        in_specs=in_specs,
        out_specs=out_specs,
        out_shape=out_shapes,
    )(x, st, W1, U1, b1.reshape(1, -1), W2, U2, b2.reshape(1, -1),
      Wm1, bm1.reshape(1, -1), Wm2, bm2.reshape(1, 1))


def _shard_filter(obs_l, k1d, k2d, W1, U1, b1, W2, U2, b2, Wm1, bm1, Wm2, bm2):
    Pn, D, B = N_PARTICLES, DIM_STATE, BATCH
    b0 = jax.lax.axis_index("x") * B_LOC

    st = jnp.zeros((B_LOC, Pn, 4 * D), jnp.float32)
    w = jnp.ones((B_LOC, Pn), jnp.float32) / Pn
    obs_t = jnp.transpose(obs_l, (1, 0, 2))
    b0_arr = b0.astype(jnp.int32).reshape(1, 1)

    np_idx = jax.lax.broadcasted_iota(jnp.uint32, (B_LOC, Pn, D), 0)
    np_p = jax.lax.broadcasted_iota(jnp.uint32, (B_LOC, Pn, D), 1)
    np_d = jax.lax.broadcasted_iota(jnp.uint32, (B_LOC, Pn, D), 2)
    noise_idx = (((np_idx + b0.astype(jnp.uint32)) * jnp.uint32(Pn) + np_p)
                 << jnp.uint32(5)) + np_d

    def step(carry, xs):
        st, w = carry
        ob, k1, k2 = xs
        idx = _cat_pallas(w, k1.reshape(1, 2), b0_arr).T
        st_g = jnp.take_along_axis(st, idx[..., None], axis=1)
        nbits = _threefry_xor(k2[0], k2[1], noise_idx)
        nf = _bits_to_unit(nbits)
        nu = jnp.maximum(_LO, nf * (jnp.float32(1.0) - _LO) + _LO)
        noise = jnp.sqrt(jnp.float32(2.0)) * jax.lax.erf_inv(nu)
        ob_t = jnp.broadcast_to(ob[:, None, :], (B_LOC, Pn, DIM_OBS))
        x = jnp.concatenate([noise, ob_t], axis=-1).reshape(ROWS_L, D + DIM_OBS)
        sto, wv = _step_pallas(x, st_g.reshape(ROWS_L, 4 * D),
                               W1, U1, b1, W2, U2, b2, Wm1, bm1, Wm2, bm2)
        w_new = wv[:, 0].reshape(B_LOC, Pn)
        return (sto.reshape(B_LOC, Pn, 4 * D), w_new), None

    (st, w), _ = jax.lax.scan(step, (st, w), (obs_t, k1d, k2d))
    return st[..., 2 * DIM_STATE:3 * DIM_STATE], w


def kernel(observations, W1, U1, b1, W2, U2, b2, Wm1, bm1, Wm2, bm2):
    T = SEQ
    keys = jax.random.split(jax.random.key(42), T)
    k12 = jax.vmap(jax.random.split)(keys)
    kd = jax.random.key_data(k12).astype(jnp.uint32)
    k1d, k2d = kd[:, 0, :], kd[:, 1, :]

    mesh = jax.make_mesh((NDEV,), ("x",))
    observations = jax.reshard(
        observations, jax.NamedSharding(mesh, P("x", None, None)))
    fn = jax.shard_map(
        _shard_filter, mesh=mesh,
        in_specs=(P("x"), P(), P(), P(), P(), P(), P(), P(), P(), P(), P(), P(), P()),
        out_specs=(P("x"), P("x")),
        check_vma=False,
    )
    return fn(observations, k1d, k2d, W1, U1, b1, W2, U2, b2, Wm1, bm1, Wm2, bm2)


# speedup vs baseline: 1.0167x; 1.0167x over previous
import functools

import jax
import jax.numpy as jnp
import numpy as np
from jax.experimental import pallas as pl
from jax.experimental.pallas import tpu as pltpu
from jax.sharding import PartitionSpec as P

DIM_STATE = 32
N_PARTICLES = 1024
DIM_OBS = 32
HIDDEN = 64
BATCH = 64
SEQ = 16

NDEV = 2 if jax.device_count() >= 2 else 1
B_LOC = BATCH // NDEV
ROWS_L = B_LOC * N_PARTICLES
BLK = 2048

_TINY = np.float32(np.finfo(np.float32).tiny)
_LO = np.float32(np.nextafter(np.float32(-1.0), np.float32(0.0)))


def _threefry_xor(kd0, kd1, x1):
    return _threefry_core(kd0, kd1, x1 + kd1)


def _threefry_core(ks0, ks1, x1):
    ks2 = ks0 ^ ks1 ^ jnp.uint32(0x1BD11BDA)
    x0 = jnp.zeros_like(x1) + ks0
    rots = ((13, 15, 26, 6), (17, 29, 16, 24))
    ks = (ks0, ks1, ks2)

    def rotl(x, d):
        return (x << jnp.uint32(d)) | (x >> jnp.uint32(32 - d))

    for i in range(5):
        for r in rots[i % 2]:
            x0 = x0 + x1
            x1 = rotl(x1, r)
            x1 = x0 ^ x1
        x0 = x0 + ks[(i + 1) % 3]
        x1 = x1 + ks[(i + 2) % 3] + jnp.uint32(i + 1)
    return x0 ^ x1


def _bits_to_unit(bits):
    fb = (bits >> jnp.uint32(9)) | jnp.uint32(0x3F800000)
    return jax.lax.bitcast_convert_type(fb, jnp.float32) - jnp.float32(1.0)


PBLK = 128
PCHUNK = 4


def _cat_kernel(k1_ref, b0_ref, w_ref, idx_ref):
    i = pl.program_id(0)
    wv = w_ref[...]
    ks0 = k1_ref[0, 0]
    ks1 = k1_ref[0, 1]
    b0 = b0_ref[0, 0]
    Pn = N_PARTICLES

    bb = jax.lax.broadcasted_iota(jnp.uint32, (PCHUNK, B_LOC, Pn), 1)
    pp = jax.lax.broadcasted_iota(jnp.uint32, (PCHUNK, B_LOC, Pn), 0)
    jj = jax.lax.broadcasted_iota(jnp.uint32, (PCHUNK, B_LOC, Pn), 2)
    jn = jax.lax.broadcasted_iota(jnp.int32, (PCHUNK, B_LOC, Pn), 2)
    b0u = b0.astype(jnp.uint32)
    inv = (((pp * jnp.uint32(BATCH) + bb + b0u) << jnp.uint32(10)) + jj) + ks1

    def body(c, _):
        base_p = (i * PBLK + c * PCHUNK).astype(jnp.uint32)
        off = base_p * jnp.uint32(BATCH * N_PARTICLES)
        bits = _threefry_core(ks0, ks1, inv + off)
        f = _bits_to_unit(bits)
        u = f * (jnp.float32(1.0) - _TINY) + _TINY
        val = -jnp.log(-jnp.log(u)) + wv[None, :, :]
        m = jnp.max(val, axis=2, keepdims=True)
        cand = jnp.where(val == m, jn, jnp.int32(Pn))
        idxp = jnp.min(cand, axis=2).astype(jnp.int32)
        idx_ref[pl.ds(c * PCHUNK, PCHUNK), :] = idxp
        return 0

    jax.lax.fori_loop(0, PBLK // PCHUNK, body, 0)


def _cat_pallas(w, k1, b0):
    return pl.pallas_call(
        _cat_kernel,
        grid=(N_PARTICLES // PBLK,),
        in_specs=[
            pl.BlockSpec(memory_space=pltpu.SMEM),
            pl.BlockSpec(memory_space=pltpu.SMEM),
            pl.BlockSpec((B_LOC, N_PARTICLES), lambda i: (0, 0)),
        ],
        out_specs=pl.BlockSpec((PBLK, B_LOC), lambda i: (i, 0)),
        out_shape=jax.ShapeDtypeStruct((N_PARTICLES, B_LOC), jnp.int32),
    )(k1, b0, w)


def _step_kernel(x_ref, st_ref,
                 W1_ref, U1_ref, b1_ref, W2_ref, U2_ref, b2_ref,
                 Wm1_ref, bm1_ref, Wm2_ref, bm2_ref,
                 sto_ref, wo_ref):
    x = x_ref[...]
    st = st_ref[...]
    D = DIM_STATE

    def bdot(a, bmat):
        return jnp.dot(a.astype(jnp.bfloat16), bmat.astype(jnp.bfloat16),
                       preferred_element_type=jnp.float32)

    def lstm(xv, h, c, W, U, b):
        z = bdot(xv, W)
        z = z + bdot(h, U)
        z = z + b[None, :]
        i = jax.nn.sigmoid(z[:, 0:D])
        f = jax.nn.sigmoid(z[:, D:2 * D])
        g = jnp.tanh(z[:, 2 * D:3 * D])
        o = jax.nn.sigmoid(z[:, 3 * D:4 * D])
        c_new = f * c + i * g
        h_new = o * jnp.tanh(c_new)
        return h_new, c_new

    h1f, c1f = lstm(x, st[:, 0:D], st[:, D:2 * D], W1_ref[...], U1_ref[...], b1_ref[...][0])
    h2f, c2f = lstm(h1f, st[:, 2 * D:3 * D], st[:, 3 * D:4 * D],
                    W2_ref[...], U2_ref[...], b2_ref[...][0])
    sto_ref[...] = jnp.concatenate([h1f, c1f, h2f, c2f], axis=1)
    ob = x[:, D:D + DIM_OBS]
    minp = jnp.concatenate([ob, h2f], axis=1)
    hid = jnp.dot(minp.astype(jnp.bfloat16), Wm1_ref[...].astype(jnp.bfloat16),
                  preferred_element_type=jnp.float32)
    hid = jax.nn.relu(hid + bm1_ref[...][0])
    wv = jnp.dot(hid.astype(jnp.bfloat16), Wm2_ref[...].astype(jnp.bfloat16),
                 preferred_element_type=jnp.float32)
    wo_ref[...] = wv + bm2_ref[...][0, 0]


def _row_spec(width):
    return pl.BlockSpec((BLK, width), lambda i: (i, 0))


def _full_spec(shape):
    return pl.BlockSpec(shape, lambda i: tuple(0 for _ in shape))


def _step_pallas(x, st, W1, U1, b1, W2, U2, b2, Wm1, bm1, Wm2, bm2):
    n = ROWS_L // BLK
    out_shapes = [jax.ShapeDtypeStruct((ROWS_L, 4 * DIM_STATE), jnp.float32),
                  jax.ShapeDtypeStruct((ROWS_L, 1), jnp.float32)]
    in_specs = [
        _row_spec(DIM_STATE + DIM_OBS),
        _row_spec(4 * DIM_STATE),
        _full_spec(W1.shape), _full_spec(U1.shape), _full_spec((1, 4 * DIM_STATE)),
        _full_spec(W2.shape), _full_spec(U2.shape), _full_spec((1, 4 * DIM_STATE)),
        _full_spec(Wm1.shape), _full_spec((1, HIDDEN)),
        _full_spec(Wm2.shape), _full_spec((1, 1)),
    ]
    out_specs = [_row_spec(4 * DIM_STATE), _row_spec(1)]
    return pl.pallas_call(
        _step_kernel,
        grid=(n,),
        in_specs=in_specs,
        out_specs=out_specs,
        out_shape=out_shapes,
    )(x, st, W1, U1, b1.reshape(1, -1), W2, U2, b2.reshape(1, -1),
      Wm1, bm1.reshape(1, -1), Wm2, bm2.reshape(1, 1))


def _shard_filter(obs_l, k1d, k2d, W1, U1, b1, W2, U2, b2, Wm1, bm1, Wm2, bm2):
    b0 = jax.lax.axis_index("x") * B_LOC
    return _filter_local(b0, obs_l, k1d, k2d, W1, U1, b1, W2, U2, b2,
                         Wm1, bm1, Wm2, bm2)


def _filter_local(b0, obs_l, k1d, k2d, W1, U1, b1, W2, U2, b2, Wm1, bm1, Wm2, bm2):
    Pn, D, B = N_PARTICLES, DIM_STATE, BATCH

    st = jnp.zeros((B_LOC, Pn, 4 * D), jnp.float32)
    w = jnp.ones((B_LOC, Pn), jnp.float32) / Pn
    obs_t = jnp.transpose(obs_l, (1, 0, 2))
    b0_arr = b0.astype(jnp.int32).reshape(1, 1)

    np_idx = jax.lax.broadcasted_iota(jnp.uint32, (B_LOC, Pn, D), 0)
    np_p = jax.lax.broadcasted_iota(jnp.uint32, (B_LOC, Pn, D), 1)
    np_d = jax.lax.broadcasted_iota(jnp.uint32, (B_LOC, Pn, D), 2)
    noise_idx = (((np_idx + b0.astype(jnp.uint32)) * jnp.uint32(Pn) + np_p)
                 << jnp.uint32(5)) + np_d

    def step(carry, xs):
        st, w = carry
        ob, k1, k2 = xs
        idx = _cat_pallas(w, k1.reshape(1, 2), b0_arr).T
        st_g = jnp.take_along_axis(st, idx[..., None], axis=1)
        nbits = _threefry_xor(k2[0], k2[1], noise_idx)
        nf = _bits_to_unit(nbits)
        nu = jnp.maximum(_LO, nf * (jnp.float32(1.0) - _LO) + _LO)
        noise = jnp.sqrt(jnp.float32(2.0)) * jax.lax.erf_inv(nu)
        ob_t = jnp.broadcast_to(ob[:, None, :], (B_LOC, Pn, DIM_OBS))
        x = jnp.concatenate([noise, ob_t], axis=-1).reshape(ROWS_L, D + DIM_OBS)
        sto, wv = _step_pallas(x, st_g.reshape(ROWS_L, 4 * D),
                               W1, U1, b1, W2, U2, b2, Wm1, bm1, Wm2, bm2)
        w_new = wv[:, 0].reshape(B_LOC, Pn)
        return (sto.reshape(B_LOC, Pn, 4 * D), w_new), None

    (st, w), _ = jax.lax.scan(step, (st, w), (obs_t, k1d, k2d))
    return st[..., 2 * DIM_STATE:3 * DIM_STATE], w


def kernel(observations, W1, U1, b1, W2, U2, b2, Wm1, bm1, Wm2, bm2):
    T = SEQ
    keys = jax.random.split(jax.random.key(42), T)
    k12 = jax.vmap(jax.random.split)(keys)
    kd = jax.random.key_data(k12).astype(jnp.uint32)
    k1d, k2d = kd[:, 0, :], kd[:, 1, :]

    if NDEV == 1:
        return _filter_local(jnp.int32(0), observations, k1d, k2d,
                             W1, U1, b1, W2, U2, b2, Wm1, bm1, Wm2, bm2)
    mesh = jax.make_mesh((NDEV,), ("x",))
    observations = jax.reshard(
        observations, jax.NamedSharding(mesh, P("x", None, None)))
    fn = jax.shard_map(
        _shard_filter, mesh=mesh,
        in_specs=(P("x"), P(), P(), P(), P(), P(), P(), P(), P(), P(), P(), P(), P()),
        out_specs=(P("x"), P("x")),
        check_vma=False,
    )
    return fn(observations, k1d, k2d, W1, U1, b1, W2, U2, b2, Wm1, bm1, Wm2, bm2)
